# baseline (speedup 1.0000x reference)
import jax
import jax.numpy as jnp
from jax import lax
from jax.experimental import pallas as pl
from jax.experimental.pallas import tpu as pltpu
from jax.experimental.pallas import tpu_sc as plsc

D_MODEL = 128
NUM_WORKERS = 32
CHUNK = 128
DEPTH = 3


def _body(x_hbm, idx_hbm, pe_hbm, out_hbm,
          xb0, xb1, xb2, pr0, pr1, pr2, idxbuf,
          seml, semg, sems):
    xbufs = (xb0, xb1, xb2)
    prows = (pr0, pr1, pr2)
    n_rows = x_hbm.shape[0]
    rows_per_worker = n_rows // NUM_WORKERS
    n_chunks = rows_per_worker // CHUNK
    wid = lax.axis_index("s") * 2 + lax.axis_index("c")
    base = wid * rows_per_worker

    def start_load(g, s):
        pltpu.async_copy(x_hbm.at[pl.ds(base + g * CHUNK, CHUNK), :],
                         xbufs[s], seml.at[s])

    def wait_load(g, s):
        pltpu.make_async_copy(x_hbm.at[pl.ds(base + g * CHUNK, CHUNK), :],
                              xbufs[s], seml.at[s]).wait()

    def start_gather(g, s):
        pltpu.async_copy(pe_hbm.at[idxbuf.at[g]], prows[s], semg.at[s])

    def wait_gather(g, s):
        pltpu.make_async_copy(pe_hbm.at[idxbuf.at[g]], prows[s],
                              semg.at[s]).wait()

    def start_store(g, s):
        pltpu.async_copy(prows[s], out_hbm.at[pl.ds(base + g * CHUNK, CHUNK), :],
                         sems.at[s])

    def wait_store(g, s):
        pltpu.make_async_copy(prows[s],
                              out_hbm.at[pl.ds(base + g * CHUNK, CHUNK), :],
                              sems.at[s]).wait()

    pltpu.sync_copy(idx_hbm.at[pl.ds(wid * n_chunks, n_chunks), :], idxbuf)

    for s in range(DEPTH):
        start_load(s, s)
    start_gather(0, 0)

    n_steps = -(-n_chunks // DEPTH)

    def step(i, carry):
        for s in range(DEPTH):
            g = i * DEPTH + s

            @pl.when(g < n_chunks)
            def _():
                wait_load(g, s)
                wait_gather(g, s)

                @plsc.parallel_loop(0, CHUNK, step=1, unroll=8)
                def add_row(r):
                    for c in range(D_MODEL // 16):
                        v = xbufs[s][r, pl.ds(c * 16, 16)]
                        plsc.addupdate(prows[s].at[r, pl.ds(c * 16, 16)], v)

                start_store(g, s)

                @pl.when(g + DEPTH < n_chunks)
                def _():
                    start_load(g + DEPTH, s)

                s1 = (s + 1) % DEPTH

                @pl.when(jnp.logical_and(g >= DEPTH - 1,
                                         g + 1 < n_chunks))
                def _():
                    wait_store(g + 1 - DEPTH, s1)

                @pl.when(g + 1 < n_chunks)
                def _():
                    start_gather(g + 1, s1)

        return carry

    lax.fori_loop(0, n_steps, step, 0)

    for k in range(DEPTH):
        g = n_chunks - DEPTH + k
        wait_store(g, g % DEPTH)


def kernel(x, frame_indices, pe):
    b, l, d = x.shape
    n = b * l
    x2 = x.reshape(n, d)
    idx2 = frame_indices.astype(jnp.int32).reshape(n // CHUNK, CHUNK)
    n_chunks = (n // NUM_WORKERS) // CHUNK

    mesh = plsc.VectorSubcoreMesh(core_axis_name="c", subcore_axis_name="s")
    run = pl.kernel(
        _body,
        out_type=jax.ShapeDtypeStruct((n, d), jnp.float32),
        mesh=mesh,
        scratch_types=(
            [pltpu.VMEM((CHUNK, D_MODEL), jnp.float32) for _ in range(DEPTH)]
            + [pltpu.VMEM((CHUNK, D_MODEL), jnp.float32) for _ in range(DEPTH)]
            + [pltpu.VMEM((n_chunks, CHUNK), jnp.int32)]
            + [pltpu.SemaphoreType.DMA((DEPTH,)),
               pltpu.SemaphoreType.DMA((DEPTH,)),
               pltpu.SemaphoreType.DMA((DEPTH,))]
        ),
    )
    out = run(x2, idx2, pe)
    return out.reshape(b, l, d)


# speedup vs baseline: 1.7327x; 1.7327x over previous
import jax
import jax.numpy as jnp
from jax import lax
from jax.experimental import pallas as pl
from jax.experimental.pallas import tpu as pltpu
from jax.experimental.pallas import tpu_sc as plsc

D_MODEL = 128
NUM_WORKERS = 32
NUM_SUBCORES = 16
CHUNK = 128
DEPTH = 3
PE_ROWS = 512
PE_PIECE = PE_ROWS // NUM_SUBCORES


def _body(x_hbm, idx_hbm, pe_hbm, out_hbm,
          xb0, xb1, xb2, pr0, pr1, pr2, idxbuf, pe_sh,
          seml, semg, sems):
    xbufs = (xb0, xb1, xb2)
    prows = (pr0, pr1, pr2)
    n_rows = x_hbm.shape[0]
    rows_per_worker = n_rows // NUM_WORKERS
    n_chunks = rows_per_worker // CHUNK
    sid = lax.axis_index("s")
    wid = sid * 2 + lax.axis_index("c")
    base = wid * rows_per_worker

    def start_load(g, s):
        pltpu.async_copy(x_hbm.at[pl.ds(base + g * CHUNK, CHUNK), :],
                         xbufs[s], seml.at[s])

    def wait_load(g, s):
        pltpu.make_async_copy(x_hbm.at[pl.ds(base + g * CHUNK, CHUNK), :],
                              xbufs[s], seml.at[s]).wait()

    def start_gather(g, s):
        pltpu.async_copy(pe_sh.at[idxbuf.at[g]], prows[s], semg.at[s])

    def wait_gather(g, s):
        pltpu.make_async_copy(pe_sh.at[idxbuf.at[g]], prows[s],
                              semg.at[s]).wait()

    def start_store(g, s):
        pltpu.async_copy(prows[s], out_hbm.at[pl.ds(base + g * CHUNK, CHUNK), :],
                         sems.at[s])

    def wait_store(g, s):
        pltpu.make_async_copy(prows[s],
                              out_hbm.at[pl.ds(base + g * CHUNK, CHUNK), :],
                              sems.at[s]).wait()

    pe_stage = pr0.at[pl.ds(0, PE_PIECE), :]
    pltpu.sync_copy(pe_hbm.at[pl.ds(sid * PE_PIECE, PE_PIECE), :], pe_stage)
    pltpu.sync_copy(pe_stage, pe_sh.at[pl.ds(sid * PE_PIECE, PE_PIECE), :])
    plsc.subcore_barrier()

    pltpu.sync_copy(idx_hbm.at[pl.ds(wid * n_chunks, n_chunks), :], idxbuf)

    for s in range(DEPTH):
        start_load(s, s)
    start_gather(0, 0)

    n_steps = -(-n_chunks // DEPTH)

    def step(i, carry):
        for s in range(DEPTH):
            g = i * DEPTH + s

            @pl.when(g < n_chunks)
            def _():
                wait_load(g, s)
                wait_gather(g, s)

                @plsc.parallel_loop(0, CHUNK, step=1, unroll=8)
                def add_row(r):
                    for c in range(D_MODEL // 16):
                        v = xbufs[s][r, pl.ds(c * 16, 16)]
                        plsc.addupdate(prows[s].at[r, pl.ds(c * 16, 16)], v)

                start_store(g, s)

                @pl.when(g + DEPTH < n_chunks)
                def _():
                    start_load(g + DEPTH, s)

                s1 = (s + 1) % DEPTH

                @pl.when(jnp.logical_and(g >= DEPTH - 1,
                                         g + 1 < n_chunks))
                def _():
                    wait_store(g + 1 - DEPTH, s1)

                @pl.when(g + 1 < n_chunks)
                def _():
                    start_gather(g + 1, s1)

        return carry

    lax.fori_loop(0, n_steps, step, 0)

    for k in range(DEPTH):
        g = n_chunks - DEPTH + k
        wait_store(g, g % DEPTH)


def kernel(x, frame_indices, pe):
    b, l, d = x.shape
    n = b * l
    x2 = x.reshape(n, d)
    idx2 = frame_indices.astype(jnp.int32).reshape(n // CHUNK, CHUNK)
    pe_pad = jnp.pad(pe, ((0, PE_ROWS - pe.shape[0]), (0, 0)))
    n_chunks = (n // NUM_WORKERS) // CHUNK

    mesh = plsc.VectorSubcoreMesh(core_axis_name="c", subcore_axis_name="s")
    run = pl.kernel(
        _body,
        out_type=jax.ShapeDtypeStruct((n, d), jnp.float32),
        mesh=mesh,
        scratch_types=(
            [pltpu.VMEM((CHUNK, D_MODEL), jnp.float32) for _ in range(DEPTH)]
            + [pltpu.VMEM((CHUNK, D_MODEL), jnp.float32) for _ in range(DEPTH)]
            + [pltpu.VMEM((n_chunks, CHUNK), jnp.int32)]
            + [pltpu.VMEM_SHARED((PE_ROWS, D_MODEL), jnp.float32)]
            + [pltpu.SemaphoreType.DMA((DEPTH,)),
               pltpu.SemaphoreType.DMA((DEPTH,)),
               pltpu.SemaphoreType.DMA((DEPTH,))]
        ),
    )
    out = run(x2, idx2, pe_pad)
    return out.reshape(b, l, d)


# speedup vs baseline: 2.2120x; 1.2766x over previous
import jax
import jax.numpy as jnp
from jax import lax
from jax.experimental import pallas as pl
from jax.experimental.pallas import tpu as pltpu
from jax.experimental.pallas import tpu_sc as plsc

D_MODEL = 128
NUM_WORKERS = 32
NUM_SUBCORES = 16
CHUNK = 128
DEPTH = 5
PE_ROWS = 512
PE_PIECE = PE_ROWS // NUM_SUBCORES


def _body(x_hbm, idx_hbm, pe_hbm, out_hbm,
          b0, b1, b2, b3, b4, idxbuf, pe_sh,
          seml, semg, sems):
    bufs = (b0, b1, b2, b3, b4)
    n_rows = x_hbm.shape[0]
    rows_per_worker = n_rows // NUM_WORKERS
    n_chunks = rows_per_worker // CHUNK
    sid = lax.axis_index("s")
    wid = sid * 2 + lax.axis_index("c")
    base = wid * rows_per_worker

    def load_slice(g):
        return x_hbm.at[pl.ds(base + g * CHUNK, CHUNK), :]

    def out_slice(g):
        return out_hbm.at[pl.ds(base + g * CHUNK, CHUNK), :]

    def start_load(g, s):
        pltpu.async_copy(load_slice(g), bufs[s], seml.at[s])

    def wait_load(g, s):
        pltpu.make_async_copy(load_slice(g), bufs[s], seml.at[s]).wait()

    def start_gather_add(g, s):
        pltpu.async_copy(pe_sh.at[idxbuf.at[g]], bufs[s], semg.at[s],
                         add=True)

    def wait_gather_add(g, s):
        pltpu.make_async_copy(pe_sh.at[idxbuf.at[g]], bufs[s],
                              semg.at[s]).wait()

    def start_store(g, s):
        pltpu.async_copy(bufs[s], out_slice(g), sems.at[s])

    def wait_store(g, s):
        pltpu.make_async_copy(bufs[s], out_slice(g), sems.at[s]).wait()

    pe_stage = b0.at[pl.ds(0, PE_PIECE), :]
    pltpu.sync_copy(pe_hbm.at[pl.ds(sid * PE_PIECE, PE_PIECE), :], pe_stage)
    pltpu.sync_copy(pe_stage, pe_sh.at[pl.ds(sid * PE_PIECE, PE_PIECE), :])
    plsc.subcore_barrier()

    pltpu.sync_copy(idx_hbm.at[pl.ds(wid * n_chunks, n_chunks), :], idxbuf)

    start_load(0, 0)
    start_load(1, 1)

    def step(i, carry):
        for s in range(DEPTH):
            g = i * DEPTH + s

            wait_load(g, s)
            start_gather_add(g, s)

            sp = (s - 1) % DEPTH

            @pl.when(g >= 1)
            def _():
                wait_gather_add(g - 1, sp)
                start_store(g - 1, sp)

            so = (s - 3) % DEPTH

            @pl.when(g >= 3)
            def _():
                wait_store(g - 3, so)

            @pl.when(g + 2 < n_chunks)
            def _():
                start_load(g + 2, (s + 2) % DEPTH)

        return carry

    lax.fori_loop(0, n_chunks // DEPTH, step, 0)

    gl = n_chunks - 1
    wait_gather_add(gl, gl % DEPTH)
    start_store(gl, gl % DEPTH)
    for g in range(n_chunks - 3, n_chunks):
        wait_store(g, g % DEPTH)


def kernel(x, frame_indices, pe):
    b, l, d = x.shape
    n = b * l
    x2 = x.reshape(n, d)
    idx2 = frame_indices.astype(jnp.int32).reshape(n // CHUNK, CHUNK)
    pe_pad = jnp.pad(pe, ((0, PE_ROWS - pe.shape[0]), (0, 0)))
    n_chunks = (n // NUM_WORKERS) // CHUNK

    mesh = plsc.VectorSubcoreMesh(core_axis_name="c", subcore_axis_name="s")
    run = pl.kernel(
        _body,
        out_type=jax.ShapeDtypeStruct((n, d), jnp.float32),
        mesh=mesh,
        scratch_types=(
            [pltpu.VMEM((CHUNK, D_MODEL), jnp.float32) for _ in range(DEPTH)]
            + [pltpu.VMEM((n_chunks, CHUNK), jnp.int32)]
            + [pltpu.VMEM_SHARED((PE_ROWS, D_MODEL), jnp.float32)]
            + [pltpu.SemaphoreType.DMA((DEPTH,)),
               pltpu.SemaphoreType.DMA((DEPTH,)),
               pltpu.SemaphoreType.DMA((DEPTH,))]
        ),
    )
    out = run(x2, idx2, pe_pad)
    return out.reshape(b, l, d)
